# baseline (speedup 1.0000x reference)
import functools

import jax
import jax.numpy as jnp
from jax import lax
from jax.experimental import pallas as pl
from jax.experimental.pallas import tpu as pltpu
from jax.experimental.pallas import tpu_sc as plsc

BB = 4
NN = 50000
KK = 128
NBR = 3
EPS_NORM = 1e-12
EPS_BN = 1e-5


_TP = 2560


def _proj_body(nrm_ref, dir_ref, proj_ref):
    d = dir_ref[...]
    ds_ = jnp.sum(d * d, axis=0, keepdims=True)
    dinv = 1.0 / jnp.maximum(jnp.sqrt(ds_), EPS_NORM)
    dnp = jnp.concatenate([d * dinv, jnp.zeros((13, KK), jnp.float32)], axis=0)

    x = nrm_ref[0]
    s = jnp.sum(x * x, axis=0, keepdims=True)
    inv = 1.0 / jnp.maximum(jnp.sqrt(s), EPS_NORM)
    xn = x * inv
    xpad = jnp.concatenate([xn, jnp.zeros((13, _TP), jnp.float32)], axis=0)
    xt = xpad.T
    proj_ref[0] = lax.dot_general(xt, dnp, (((1,), (0,)), ((), ())),
                                  preferred_element_type=jnp.float32)


_R = 160
_G = 96
_NG = (_R * NBR) // _G
_NCHUNK = (BB * NN) // _R
_NW = 32
_NV = KK // 16


def _gather_kernel(gidx_hbm, proj_hbm, feat_hbm, part_hbm,
                   idx_v, rows_v, self_v, featc_v, stat_v, sem):
    cid = lax.axis_index("c")
    sid = lax.axis_index("s")
    wid = sid * 2 + cid

    zeros16 = jnp.zeros((16,), jnp.float32)
    acc0 = [zeros16] * (2 * _NV)

    def chunk_body(t, acc):
        c = wid + t * _NW
        base = c * _R

        pltpu.sync_copy(gidx_hbm.at[pl.ds(base * NBR, _R * NBR)], idx_v)
        descs = [
            pltpu.async_copy(
                proj_hbm.at[idx_v.at[pl.ds(g * _G, _G)]],
                rows_v.at[pl.ds(g * _G, _G)],
                sem,
            )
            for g in range(_NG)
        ]
        pltpu.sync_copy(proj_hbm.at[pl.ds(base, _R)], self_v)
        for d_ in descs:
            d_.wait()

        def node_body(r, acc2):
            out = []
            for v in range(_NV):
                sl = pl.ds(v * 16, 16)
                m01 = jnp.maximum(rows_v[NBR * r, sl], rows_v[NBR * r + 1, sl])
                m2s = jnp.maximum(rows_v[NBR * r + 2, sl], self_v[r, sl])
                f = jnp.maximum(m01, m2s)
                featc_v[r, sl] = f
                out.append(acc2[v] + f)
                out.append(acc2[_NV + v] + f * f)
            return out[0::2] + out[1::2]

        acc = lax.fori_loop(0, _R, node_body, acc)
        pltpu.sync_copy(featc_v, feat_hbm.at[pl.ds(base, _R)])
        return acc

    nchunks = (_NCHUNK - wid + _NW - 1) // _NW
    acc = lax.fori_loop(0, nchunks, chunk_body, acc0)

    for v in range(_NV):
        sl = pl.ds(v * 16, 16)
        stat_v[0, sl] = acc[v]
        stat_v[1, sl] = acc[_NV + v]
    pltpu.sync_copy(stat_v, part_hbm.at[wid])


_TE = 2560


def _apply_body(f_ref, part_ref, gm_ref, bt_ref, o_ref):
    p = part_ref[...]
    cnt = float(BB * NN)
    sums = jnp.sum(p[:, 0, :], axis=0, keepdims=True)
    sqs = jnp.sum(p[:, 1, :], axis=0, keepdims=True)
    mean = sums / cnt
    var = sqs / cnt - mean * mean
    rstd = lax.rsqrt(var + EPS_BN)
    scale = gm_ref[...] * rstd
    shift = bt_ref[...] - mean * scale
    y = jnp.maximum(f_ref[0] * scale + shift, 0.0)
    o_ref[0] = y.T


def kernel(normals, ring_n, directions, gamma, beta):
    proj = pl.pallas_call(
        _proj_body,
        grid=(BB, pl.cdiv(NN, _TP)),
        in_specs=[
            pl.BlockSpec((1, 3, _TP), lambda b, i: (b, 0, i)),
            pl.BlockSpec((3, KK), lambda b, i: (0, 0)),
        ],
        out_specs=pl.BlockSpec((1, _TP, KK), lambda b, i: (b, i, 0)),
        out_shape=jax.ShapeDtypeStruct((BB, NN, KK), jnp.float32),
    )(normals, directions)

    gidx = (ring_n.astype(jnp.int32)
            + (jnp.arange(BB, dtype=jnp.int32) * NN)[:, None, None]
            ).reshape(BB * NN * NBR)
    proj_flat = proj.reshape(BB * NN, KK)

    mesh = plsc.VectorSubcoreMesh(core_axis_name="c", subcore_axis_name="s")
    feat, partials = functools.partial(
        pl.kernel,
        mesh=mesh,
        compiler_params=pltpu.CompilerParams(
            use_tc_tiling_on_sc=True, needs_layout_passes=False),
        out_type=[
            jax.ShapeDtypeStruct((BB * NN, KK), jnp.float32),
            jax.ShapeDtypeStruct((_NW, 2, KK), jnp.float32),
        ],
        scratch_types=[
            pltpu.VMEM((_R * NBR,), jnp.int32),
            pltpu.VMEM((_R * NBR, KK), jnp.float32),
            pltpu.VMEM((_R, KK), jnp.float32),
            pltpu.VMEM((_R, KK), jnp.float32),
            pltpu.VMEM((2, KK), jnp.float32),
            pltpu.SemaphoreType.DMA,
        ],
    )(_gather_kernel)(gidx, proj_flat)

    feat3 = feat.reshape(BB, NN, KK)
    gm = gamma.reshape(1, KK)
    bt = beta.reshape(1, KK)

    out = pl.pallas_call(
        _apply_body,
        grid=(BB, pl.cdiv(NN, _TE)),
        in_specs=[
            pl.BlockSpec((1, _TE, KK), lambda b, i: (b, i, 0)),
            pl.BlockSpec((_NW, 2, KK), lambda b, i: (0, 0, 0)),
            pl.BlockSpec((1, KK), lambda b, i: (0, 0)),
            pl.BlockSpec((1, KK), lambda b, i: (0, 0)),
        ],
        out_specs=pl.BlockSpec((1, KK, _TE), lambda b, i: (b, 0, i)),
        out_shape=jax.ShapeDtypeStruct((BB, KK, NN), jnp.float32),
    )(feat3, partials, gm, bt)

    return out
